# baseline (speedup 1.0000x reference)
import functools

import jax
import jax.numpy as jnp
from jax import lax
from jax.experimental import pallas as pl
from jax.experimental.pallas import tpu as pltpu
from jax.experimental.pallas import tpu_sc as plsc

_LR = 0.1
_L = 16
_MASKHI = -65536
_RND = 0x8000


def _build(B, N, E):
    assert B == 4 and N % 16 == 0
    EPT1 = E // 16
    EPT2 = E // 8
    C = 2000
    NCHUNK1 = EPT1 // C
    NCHUNK2 = EPT2 // C
    assert EPT1 % C == 0 and EPT2 % C == 0
    assert NCHUNK1 % 2 == 0 and NCHUNK2 % 2 == 0
    VPC = C // _L
    NP8 = N // 8
    SIZE = NP8 + 16 - (NP8 % 16) if NP8 % 16 else NP8
    assert SIZE % 16 == 0

    mesh = plsc.VectorSubcoreMesh(core_axis_name="c", subcore_axis_name="s")

    @functools.partial(
        pl.kernel,
        mesh=mesh,
        compiler_params=pltpu.CompilerParams(needs_layout_passes=False),
        out_type=(
            jax.ShapeDtypeStruct((B * N,), jnp.float32),
            jax.ShapeDtypeStruct((2 * E,), jnp.float32),
            jax.ShapeDtypeStruct((32 * N,), jnp.float32),
        ),
        scratch_types=[
            pltpu.VMEM((N,), jnp.float32),
            pltpu.VMEM((C,), jnp.int32),
            pltpu.VMEM((C,), jnp.int32),
            pltpu.VMEM((C,), jnp.int32),
            pltpu.VMEM((C,), jnp.int32),
            pltpu.VMEM((C,), jnp.float32),
            pltpu.VMEM((C,), jnp.float32),
            pltpu.VMEM((C,), jnp.float32),
            pltpu.VMEM((C,), jnp.float32),
            pltpu.SemaphoreType.DMA((2,)),
            pltpu.SemaphoreType.DMA((2,)),
        ],
    )
    def run(x_hbm, xp_hbm, src_hbm, dst_hbm, th_hbm, z_hbm, out_hbm, cur_hbm,
            part_hbm, xacc, sb0, sb1, db0, db1, tb0, tb1, cb0, cb1,
            insem, outsem):
        cid = lax.axis_index("c")
        sid = lax.axis_index("s")
        grp = cid
        bg = cid * 2 + sid // 8
        half = sid // 8
        c8 = sid % 8
        ebase1 = pl.multiple_of(sid * EPT1, 8)
        ebase2 = pl.multiple_of(c8 * EPT2, 8)
        wrow = bg * 8 + c8
        sb = (sb0, sb1)
        db = (db0, db1)
        tb = (tb0, tb1)
        cb = (cb0, cb1)

        def coff1(ci):
            return pl.multiple_of(grp * E + ebase1 + ci * C, 8)

        def coff2(ci):
            return pl.multiple_of(grp * E + ebase2 + ci * C, 8)

        def issue_in1(ci, slot):
            base = pl.multiple_of(ebase1 + ci * C, 8)
            pltpu.async_copy(src_hbm.at[pl.ds(base, C)], sb[slot], insem.at[slot])
            pltpu.async_copy(dst_hbm.at[pl.ds(base, C)], db[slot], insem.at[slot])
            pltpu.async_copy(th_hbm.at[pl.ds(base, C)], tb[slot], insem.at[slot])

        def wait_in1(ci, slot):
            base = pl.multiple_of(ebase1 + ci * C, 8)
            pltpu.make_async_copy(src_hbm.at[pl.ds(base, C)], sb[slot],
                                  insem.at[slot]).wait()
            pltpu.make_async_copy(dst_hbm.at[pl.ds(base, C)], db[slot],
                                  insem.at[slot]).wait()
            pltpu.make_async_copy(th_hbm.at[pl.ds(base, C)], tb[slot],
                                  insem.at[slot]).wait()

        def issue_in2(ci, slot):
            base = pl.multiple_of(ebase2 + ci * C, 8)
            pltpu.async_copy(src_hbm.at[pl.ds(base, C)], sb[slot], insem.at[slot])
            pltpu.async_copy(dst_hbm.at[pl.ds(base, C)], db[slot], insem.at[slot])
            pltpu.async_copy(cur_hbm.at[pl.ds(coff2(ci), C)], tb[slot],
                             insem.at[slot])

        def wait_in2(ci, slot):
            base = pl.multiple_of(ebase2 + ci * C, 8)
            pltpu.make_async_copy(src_hbm.at[pl.ds(base, C)], sb[slot],
                                  insem.at[slot]).wait()
            pltpu.make_async_copy(dst_hbm.at[pl.ds(base, C)], db[slot],
                                  insem.at[slot]).wait()
            pltpu.make_async_copy(cur_hbm.at[pl.ds(coff2(ci), C)], tb[slot],
                                  insem.at[slot]).wait()

        scope1 = jax.named_scope("ph1_currents")
        scope1.__enter__()
        xcp = pltpu.async_copy(
            xp_hbm.at[pl.ds(pl.multiple_of(grp * N, 8), N)], xacc, insem.at[0])
        xcp.wait()
        issue_in1(0, 0)

        def compute1(slot):
            sbuf, dbuf, gbuf, cbuf = sb[slot], db[slot], tb[slot], cb[slot]

            @plsc.parallel_loop(0, VPC, unroll=4)
            def _(i):
                o = i * _L
                sv = sbuf[pl.ds(o, _L)]
                dv = dbuf[pl.ds(o, _L)]
                g = gbuf[pl.ds(o, _L)]
                ps = plsc.bitcast(plsc.load_gather(xacc, [sv]), jnp.int32)
                pd = plsc.bitcast(plsc.load_gather(xacc, [dv]), jnp.int32)
                xs0 = plsc.bitcast(ps << 16, jnp.float32)
                xd0 = plsc.bitcast(pd << 16, jnp.float32)
                xs1 = plsc.bitcast(ps & _MASKHI, jnp.float32)
                xd1 = plsc.bitcast(pd & _MASKHI, jnp.float32)
                c0 = g * (xs0 - xd0)
                c1 = g * (xs1 - xd1)
                b0 = plsc.bitcast(c0, jnp.int32)
                b1 = plsc.bitcast(c1, jnp.int32)
                lo = lax.shift_right_logical(b0 + _RND, 16)
                hi = (b1 + _RND) & _MASKHI
                cbuf[pl.ds(o, _L)] = plsc.bitcast(lo | hi, jnp.float32)

        def chunk1(ci2, _):
            ci = ci2 * 2
            issue_in1(ci + 1, 1)
            wait_in1(ci, 0)

            @pl.when(ci2 >= 1)
            def _():
                pltpu.make_async_copy(cb0, cur_hbm.at[pl.ds(coff1(ci), C)],
                                      outsem.at[0]).wait()

            compute1(0)
            pltpu.async_copy(cb0, cur_hbm.at[pl.ds(coff1(ci), C)], outsem.at[0])

            @pl.when(ci + 2 < NCHUNK1)
            def _():
                issue_in1(ci + 2, 0)

            wait_in1(ci + 1, 1)

            @pl.when(ci2 >= 1)
            def _():
                pltpu.make_async_copy(cb1, cur_hbm.at[pl.ds(coff1(ci + 1), C)],
                                      outsem.at[1]).wait()

            compute1(1)
            pltpu.async_copy(cb1, cur_hbm.at[pl.ds(coff1(ci + 1), C)],
                             outsem.at[1])
            return 0

        lax.fori_loop(0, NCHUNK1 // 2, chunk1, 0)
        pltpu.make_async_copy(cb0, cur_hbm.at[pl.ds(coff1(NCHUNK1 - 2), C)],
                              outsem.at[0]).wait()
        pltpu.make_async_copy(cb1, cur_hbm.at[pl.ds(coff1(NCHUNK1 - 1), C)],
                              outsem.at[1]).wait()
        plsc.subcore_barrier()
        scope1.__exit__(None, None, None)
        scope2 = jax.named_scope("ph2_scatter")
        scope2.__enter__()

        issue_in2(0, 0)
        zcp = pltpu.async_copy(z_hbm, xacc, outsem.at[0])
        zcp.wait()
        shv = jnp.zeros((_L,), jnp.int32) + half * 16

        def compute2(slot):
            sbuf, dbuf, cbuf = sb[slot], db[slot], tb[slot]

            @plsc.parallel_loop(0, VPC, unroll=8)
            def _(i):
                o = i * _L
                sv = sbuf[pl.ds(o, _L)]
                dv = dbuf[pl.ds(o, _L)]
                cp = plsc.bitcast(cbuf[pl.ds(o, _L)], jnp.int32)
                cv = plsc.bitcast(lax.shift_right_arithmetic(cp, shv) << 16,
                                  jnp.float32)
                plsc.addupdate_scatter(xacc, [sv], cv)
                plsc.addupdate_scatter(xacc, [dv], -cv)

        def chunk2(ci2, _):
            ci = ci2 * 2
            issue_in2(ci + 1, 1)
            wait_in2(ci, 0)
            compute2(0)

            @pl.when(ci + 2 < NCHUNK2)
            def _():
                issue_in2(ci + 2, 0)

            wait_in2(ci + 1, 1)
            compute2(1)
            return 0

        lax.fori_loop(0, NCHUNK2 // 2, chunk2, 0)

        scope2.__exit__(None, None, None)
        scope3 = jax.named_scope("ph3_combine")
        scope3.__enter__()
        pltpu.sync_copy(xacc, part_hbm.at[pl.ds(pl.multiple_of(wrow * N, 8), N)])
        plsc.subcore_barrier()

        nstart = c8 * NP8 - (c8 * NP8) % 16
        nstart = pl.multiple_of(lax.min(nstart, N - SIZE), 16)
        xo = 0
        po = (SIZE, 2 * SIZE)

        def ppart(j):
            return part_hbm.at[pl.ds(
                pl.multiple_of((bg * 8 + j) * N + nstart, 16), SIZE)]

        xslice = x_hbm.at[pl.ds(pl.multiple_of(bg * N + nstart, 16), SIZE)]
        pltpu.async_copy(xslice, xacc.at[pl.ds(xo, SIZE)], insem.at[0])
        pltpu.async_copy(ppart(0), xacc.at[pl.ds(po[0], SIZE)], outsem.at[0])
        pltpu.make_async_copy(xslice, xacc.at[pl.ds(xo, SIZE)],
                              insem.at[0]).wait()
        for j in range(8):
            slot = j % 2
            if j + 1 < 8:
                pltpu.async_copy(ppart(j + 1), xacc.at[pl.ds(po[1 - slot], SIZE)],
                                 outsem.at[1 - slot])
            pltpu.make_async_copy(ppart(j), xacc.at[pl.ds(po[slot], SIZE)],
                                  outsem.at[slot]).wait()
            pj = po[slot]

            @plsc.parallel_loop(0, SIZE // _L, unroll=4)
            def _(i, _pj=pj):
                o = i * _L
                xacc[pl.ds(xo + o, _L)] = (
                    xacc[pl.ds(xo + o, _L)] - _LR * xacc[pl.ds(_pj + o, _L)])

        pltpu.sync_copy(xacc.at[pl.ds(xo, SIZE)],
                        out_hbm.at[pl.ds(pl.multiple_of(bg * N + nstart, 16), SIZE)])
        scope3.__exit__(None, None, None)

    return run


def kernel(x, theta, edge_index):
    B, N = x.shape
    E = edge_index.shape[1]
    src = edge_index[0].astype(jnp.int32)
    dst = edge_index[1].astype(jnp.int32)
    th = theta.reshape(E)
    xf = x.reshape(B * N)
    xb = lax.bitcast_convert_type(x.astype(jnp.bfloat16), jnp.uint16)
    xb = xb.astype(jnp.uint32)
    xp = jnp.concatenate([xb[0] | (xb[1] << 16), xb[2] | (xb[3] << 16)])
    xp = lax.bitcast_convert_type(xp, jnp.float32)
    zeros_n = jnp.zeros((N,), jnp.float32)
    out_flat, _, _ = _build(B, N, E)(xf, xp, src, dst, th, zeros_n)
    return out_flat.reshape(B, N)


# speedup vs baseline: 1.0016x; 1.0016x over previous
import functools

import jax
import jax.numpy as jnp
from jax import lax
from jax.experimental import pallas as pl
from jax.experimental.pallas import tpu as pltpu
from jax.experimental.pallas import tpu_sc as plsc

_LR = 0.1
_L = 16
_MASKHI = -65536
_RND = 0x8000


def _build(B, N, E):
    assert B == 4 and N % 16 == 0
    EPT1 = E // 16
    EPT2 = E // 8
    C = 2000
    NCHUNK1 = EPT1 // C
    NCHUNK2 = EPT2 // C
    assert EPT1 % C == 0 and EPT2 % C == 0
    assert NCHUNK1 % 2 == 0 and NCHUNK2 % 2 == 0
    VPC = C // _L
    NP8 = N // 8
    SIZE = NP8 + 16 - (NP8 % 16) if NP8 % 16 else NP8
    assert SIZE % 16 == 0

    mesh = plsc.VectorSubcoreMesh(core_axis_name="c", subcore_axis_name="s")

    @functools.partial(
        pl.kernel,
        mesh=mesh,
        compiler_params=pltpu.CompilerParams(needs_layout_passes=False),
        out_type=(
            jax.ShapeDtypeStruct((B * N,), jnp.float32),
            jax.ShapeDtypeStruct((2 * E,), jnp.float32),
            jax.ShapeDtypeStruct((32 * N,), jnp.float32),
        ),
        scratch_types=[
            pltpu.VMEM((N,), jnp.float32),
            pltpu.VMEM((C,), jnp.int32),
            pltpu.VMEM((C,), jnp.int32),
            pltpu.VMEM((C,), jnp.int32),
            pltpu.VMEM((C,), jnp.int32),
            pltpu.VMEM((C,), jnp.float32),
            pltpu.VMEM((C,), jnp.float32),
            pltpu.VMEM((C,), jnp.float32),
            pltpu.VMEM((C,), jnp.float32),
            pltpu.SemaphoreType.DMA((2,)),
            pltpu.SemaphoreType.DMA((2,)),
        ],
    )
    def run(x_hbm, xp_hbm, src_hbm, dst_hbm, th_hbm, z_hbm, out_hbm, cur_hbm,
            part_hbm, xacc, sb0, sb1, db0, db1, tb0, tb1, cb0, cb1,
            insem, outsem):
        cid = lax.axis_index("c")
        sid = lax.axis_index("s")
        grp = cid
        bg = cid * 2 + sid // 8
        half = sid // 8
        c8 = sid % 8
        ebase1 = pl.multiple_of(sid * EPT1, 8)
        ebase2 = pl.multiple_of(c8 * EPT2, 8)
        wrow = bg * 8 + c8
        sb = (sb0, sb1)
        db = (db0, db1)
        tb = (tb0, tb1)
        cb = (cb0, cb1)

        def coff1(ci):
            return pl.multiple_of(grp * E + ebase1 + ci * C, 8)

        def coff2(ci):
            return pl.multiple_of(grp * E + ebase2 + ci * C, 8)

        def issue_in1(ci, slot):
            base = pl.multiple_of(ebase1 + ci * C, 8)
            pltpu.async_copy(src_hbm.at[pl.ds(base, C)], sb[slot], insem.at[slot])
            pltpu.async_copy(dst_hbm.at[pl.ds(base, C)], db[slot], insem.at[slot])
            pltpu.async_copy(th_hbm.at[pl.ds(base, C)], tb[slot], insem.at[slot])

        def wait_in1(ci, slot):
            base = pl.multiple_of(ebase1 + ci * C, 8)
            pltpu.make_async_copy(src_hbm.at[pl.ds(base, C)], sb[slot],
                                  insem.at[slot]).wait()
            pltpu.make_async_copy(dst_hbm.at[pl.ds(base, C)], db[slot],
                                  insem.at[slot]).wait()
            pltpu.make_async_copy(th_hbm.at[pl.ds(base, C)], tb[slot],
                                  insem.at[slot]).wait()

        def issue_in2(ci, slot):
            base = pl.multiple_of(ebase2 + ci * C, 8)
            pltpu.async_copy(src_hbm.at[pl.ds(base, C)], sb[slot], insem.at[slot])
            pltpu.async_copy(dst_hbm.at[pl.ds(base, C)], db[slot], insem.at[slot])
            pltpu.async_copy(cur_hbm.at[pl.ds(coff2(ci), C)], tb[slot],
                             insem.at[slot])

        def wait_in2(ci, slot):
            base = pl.multiple_of(ebase2 + ci * C, 8)
            pltpu.make_async_copy(src_hbm.at[pl.ds(base, C)], sb[slot],
                                  insem.at[slot]).wait()
            pltpu.make_async_copy(dst_hbm.at[pl.ds(base, C)], db[slot],
                                  insem.at[slot]).wait()
            pltpu.make_async_copy(cur_hbm.at[pl.ds(coff2(ci), C)], tb[slot],
                                  insem.at[slot]).wait()

        scope1 = jax.named_scope("ph1_currents")
        scope1.__enter__()
        xcp = pltpu.async_copy(
            xp_hbm.at[pl.ds(pl.multiple_of(grp * N, 8), N)], xacc, insem.at[0])
        xcp.wait()
        issue_in1(0, 0)

        def compute1(slot):
            sbuf, dbuf, gbuf, cbuf = sb[slot], db[slot], tb[slot], cb[slot]

            @plsc.parallel_loop(0, VPC, unroll=4)
            def _(i):
                o = i * _L
                sv = sbuf[pl.ds(o, _L)]
                dv = dbuf[pl.ds(o, _L)]
                g = gbuf[pl.ds(o, _L)]
                ps = plsc.bitcast(plsc.load_gather(xacc, [sv]), jnp.int32)
                pd = plsc.bitcast(plsc.load_gather(xacc, [dv]), jnp.int32)
                xs0 = plsc.bitcast(ps << 16, jnp.float32)
                xd0 = plsc.bitcast(pd << 16, jnp.float32)
                xs1 = plsc.bitcast(ps & _MASKHI, jnp.float32)
                xd1 = plsc.bitcast(pd & _MASKHI, jnp.float32)
                c0 = g * (xs0 - xd0)
                c1 = g * (xs1 - xd1)
                b0 = plsc.bitcast(c0, jnp.int32)
                b1 = plsc.bitcast(c1, jnp.int32)
                lo = lax.shift_right_logical(b0 + _RND, 16)
                hi = (b1 + _RND) & _MASKHI
                cbuf[pl.ds(o, _L)] = plsc.bitcast(lo | hi, jnp.float32)

        def chunk1(ci2, _):
            ci = ci2 * 2
            issue_in1(ci + 1, 1)
            wait_in1(ci, 0)

            @pl.when(ci2 >= 1)
            def _():
                pltpu.make_async_copy(cb0, cur_hbm.at[pl.ds(coff1(ci), C)],
                                      outsem.at[0]).wait()

            compute1(0)
            pltpu.async_copy(cb0, cur_hbm.at[pl.ds(coff1(ci), C)], outsem.at[0])

            @pl.when(ci + 2 < NCHUNK1)
            def _():
                issue_in1(ci + 2, 0)

            wait_in1(ci + 1, 1)

            @pl.when(ci2 >= 1)
            def _():
                pltpu.make_async_copy(cb1, cur_hbm.at[pl.ds(coff1(ci + 1), C)],
                                      outsem.at[1]).wait()

            compute1(1)
            pltpu.async_copy(cb1, cur_hbm.at[pl.ds(coff1(ci + 1), C)],
                             outsem.at[1])
            return 0

        lax.fori_loop(0, NCHUNK1 // 2, chunk1, 0)
        pltpu.make_async_copy(cb0, cur_hbm.at[pl.ds(coff1(NCHUNK1 - 2), C)],
                              outsem.at[0]).wait()
        pltpu.make_async_copy(cb1, cur_hbm.at[pl.ds(coff1(NCHUNK1 - 1), C)],
                              outsem.at[1]).wait()
        plsc.subcore_barrier()
        scope1.__exit__(None, None, None)
        scope2 = jax.named_scope("ph2_scatter")
        scope2.__enter__()

        issue_in2(0, 0)
        zcp = pltpu.async_copy(z_hbm, xacc, outsem.at[0])
        zcp.wait()
        shv = jnp.zeros((_L,), jnp.int32) + half * 16

        def compute2(slot):
            sbuf, dbuf, cbuf = sb[slot], db[slot], tb[slot]

            @plsc.parallel_loop(0, VPC, unroll=4)
            def _(i):
                o = i * _L
                sv = sbuf[pl.ds(o, _L)]
                dv = dbuf[pl.ds(o, _L)]
                cp = plsc.bitcast(cbuf[pl.ds(o, _L)], jnp.int32)
                cv = plsc.bitcast(lax.shift_right_arithmetic(cp, shv) << 16,
                                  jnp.float32)
                plsc.addupdate_scatter(xacc, [sv], cv)
                plsc.addupdate_scatter(xacc, [dv], -cv)

        def chunk2(ci2, _):
            ci = ci2 * 2
            issue_in2(ci + 1, 1)
            wait_in2(ci, 0)
            compute2(0)

            @pl.when(ci + 2 < NCHUNK2)
            def _():
                issue_in2(ci + 2, 0)

            wait_in2(ci + 1, 1)
            compute2(1)
            return 0

        lax.fori_loop(0, NCHUNK2 // 2, chunk2, 0)

        scope2.__exit__(None, None, None)
        scope3 = jax.named_scope("ph3_combine")
        scope3.__enter__()
        pltpu.sync_copy(xacc, part_hbm.at[pl.ds(pl.multiple_of(wrow * N, 8), N)])
        plsc.subcore_barrier()

        nstart = c8 * NP8 - (c8 * NP8) % 16
        nstart = pl.multiple_of(lax.min(nstart, N - SIZE), 16)
        xo = 0
        po = (SIZE, 2 * SIZE)

        def ppart(j):
            return part_hbm.at[pl.ds(
                pl.multiple_of((bg * 8 + j) * N + nstart, 16), SIZE)]

        xslice = x_hbm.at[pl.ds(pl.multiple_of(bg * N + nstart, 16), SIZE)]
        pltpu.async_copy(xslice, xacc.at[pl.ds(xo, SIZE)], insem.at[0])
        pltpu.async_copy(ppart(0), xacc.at[pl.ds(po[0], SIZE)], outsem.at[0])
        pltpu.make_async_copy(xslice, xacc.at[pl.ds(xo, SIZE)],
                              insem.at[0]).wait()
        for j in range(8):
            slot = j % 2
            if j + 1 < 8:
                pltpu.async_copy(ppart(j + 1), xacc.at[pl.ds(po[1 - slot], SIZE)],
                                 outsem.at[1 - slot])
            pltpu.make_async_copy(ppart(j), xacc.at[pl.ds(po[slot], SIZE)],
                                  outsem.at[slot]).wait()
            pj = po[slot]

            @plsc.parallel_loop(0, SIZE // _L, unroll=4)
            def _(i, _pj=pj):
                o = i * _L
                xacc[pl.ds(xo + o, _L)] = (
                    xacc[pl.ds(xo + o, _L)] - _LR * xacc[pl.ds(_pj + o, _L)])

        pltpu.sync_copy(xacc.at[pl.ds(xo, SIZE)],
                        out_hbm.at[pl.ds(pl.multiple_of(bg * N + nstart, 16), SIZE)])
        scope3.__exit__(None, None, None)

    return run


def kernel(x, theta, edge_index):
    B, N = x.shape
    E = edge_index.shape[1]
    src = edge_index[0].astype(jnp.int32)
    dst = edge_index[1].astype(jnp.int32)
    th = theta.reshape(E)
    xf = x.reshape(B * N)
    xb = lax.bitcast_convert_type(x.astype(jnp.bfloat16), jnp.uint16)
    xb = xb.astype(jnp.uint32)
    xp = jnp.concatenate([xb[0] | (xb[1] << 16), xb[2] | (xb[3] << 16)])
    xp = lax.bitcast_convert_type(xp, jnp.float32)
    zeros_n = jnp.zeros((N,), jnp.float32)
    out_flat, _, _ = _build(B, N, E)(xf, xp, src, dst, th, zeros_n)
    return out_flat.reshape(B, N)


# speedup vs baseline: 1.0272x; 1.0256x over previous
import functools

import jax
import jax.numpy as jnp
from jax import lax
from jax.experimental import pallas as pl
from jax.experimental.pallas import tpu as pltpu
from jax.experimental.pallas import tpu_sc as plsc

_LR = 0.1
_L = 16
_MASKHI = -65536
_RND = 0x8000


def _build(B, N, E):
    assert B == 4 and N % 16 == 0
    EPT1 = E // 16
    EPT2 = E // 8
    C = 2000
    NCHUNK1 = EPT1 // C
    NCHUNK2 = EPT2 // C
    assert EPT1 % C == 0 and EPT2 % C == 0
    assert NCHUNK1 % 2 == 0 and NCHUNK2 % 2 == 0
    VPC = C // _L
    NP8 = N // 8
    SIZE = NP8 + 16 - (NP8 % 16) if NP8 % 16 else NP8
    assert SIZE % 16 == 0

    mesh = plsc.VectorSubcoreMesh(core_axis_name="c", subcore_axis_name="s")

    @functools.partial(
        pl.kernel,
        mesh=mesh,
        compiler_params=pltpu.CompilerParams(needs_layout_passes=False),
        out_type=(
            jax.ShapeDtypeStruct((B * N,), jnp.float32),
            jax.ShapeDtypeStruct((2 * E,), jnp.float32),
            jax.ShapeDtypeStruct((32 * N,), jnp.float32),
        ),
        scratch_types=[
            pltpu.VMEM((N,), jnp.float32),
            pltpu.VMEM((C,), jnp.int32),
            pltpu.VMEM((C,), jnp.int32),
            pltpu.VMEM((C,), jnp.int32),
            pltpu.VMEM((C,), jnp.int32),
            pltpu.VMEM((C,), jnp.float32),
            pltpu.VMEM((C,), jnp.float32),
            pltpu.VMEM((C,), jnp.float32),
            pltpu.VMEM((C,), jnp.float32),
            pltpu.SemaphoreType.DMA((2,)),
            pltpu.SemaphoreType.DMA((2,)),
        ],
    )
    def run(x_hbm, xp_hbm, src_hbm, dst_hbm, th_hbm, out_hbm, cur_hbm,
            part_hbm, xacc, sb0, sb1, db0, db1, tb0, tb1, cb0, cb1,
            insem, outsem):
        cid = lax.axis_index("c")
        sid = lax.axis_index("s")
        grp = cid
        bg = cid * 2 + sid // 8
        half = sid // 8
        c8 = sid % 8
        ebase1 = pl.multiple_of(sid * EPT1, 8)
        ebase2 = pl.multiple_of(c8 * EPT2, 8)
        wrow = bg * 8 + c8
        sb = (sb0, sb1)
        db = (db0, db1)
        tb = (tb0, tb1)
        cb = (cb0, cb1)

        def coff1(ci):
            return pl.multiple_of(grp * E + ebase1 + ci * C, 8)

        def coff2(ci):
            return pl.multiple_of(grp * E + ebase2 + ci * C, 8)

        def issue_in1(ci, slot):
            base = pl.multiple_of(ebase1 + ci * C, 8)
            pltpu.async_copy(src_hbm.at[pl.ds(base, C)], sb[slot], insem.at[slot])
            pltpu.async_copy(dst_hbm.at[pl.ds(base, C)], db[slot], insem.at[slot])
            pltpu.async_copy(th_hbm.at[pl.ds(base, C)], tb[slot], insem.at[slot])

        def wait_in1(ci, slot):
            base = pl.multiple_of(ebase1 + ci * C, 8)
            pltpu.make_async_copy(src_hbm.at[pl.ds(base, C)], sb[slot],
                                  insem.at[slot]).wait()
            pltpu.make_async_copy(dst_hbm.at[pl.ds(base, C)], db[slot],
                                  insem.at[slot]).wait()
            pltpu.make_async_copy(th_hbm.at[pl.ds(base, C)], tb[slot],
                                  insem.at[slot]).wait()

        def issue_in2(ci, slot):
            base = pl.multiple_of(ebase2 + ci * C, 8)
            pltpu.async_copy(src_hbm.at[pl.ds(base, C)], sb[slot], insem.at[slot])
            pltpu.async_copy(dst_hbm.at[pl.ds(base, C)], db[slot], insem.at[slot])
            pltpu.async_copy(cur_hbm.at[pl.ds(coff2(ci), C)], tb[slot],
                             insem.at[slot])

        def wait_in2(ci, slot):
            base = pl.multiple_of(ebase2 + ci * C, 8)
            pltpu.make_async_copy(src_hbm.at[pl.ds(base, C)], sb[slot],
                                  insem.at[slot]).wait()
            pltpu.make_async_copy(dst_hbm.at[pl.ds(base, C)], db[slot],
                                  insem.at[slot]).wait()
            pltpu.make_async_copy(cur_hbm.at[pl.ds(coff2(ci), C)], tb[slot],
                                  insem.at[slot]).wait()

        scope1 = jax.named_scope("ph1_currents")
        scope1.__enter__()
        xcp = pltpu.async_copy(
            xp_hbm.at[pl.ds(pl.multiple_of(grp * N, 8), N)], xacc, insem.at[0])
        xcp.wait()
        issue_in1(0, 0)

        def compute1(slot):
            sbuf, dbuf, gbuf, cbuf = sb[slot], db[slot], tb[slot], cb[slot]

            @plsc.parallel_loop(0, VPC, unroll=4)
            def _(i):
                o = i * _L
                sv = sbuf[pl.ds(o, _L)]
                dv = dbuf[pl.ds(o, _L)]
                g = gbuf[pl.ds(o, _L)]
                ps = plsc.bitcast(plsc.load_gather(xacc, [sv]), jnp.int32)
                pd = plsc.bitcast(plsc.load_gather(xacc, [dv]), jnp.int32)
                xs0 = plsc.bitcast(ps << 16, jnp.float32)
                xd0 = plsc.bitcast(pd << 16, jnp.float32)
                xs1 = plsc.bitcast(ps & _MASKHI, jnp.float32)
                xd1 = plsc.bitcast(pd & _MASKHI, jnp.float32)
                c0 = g * (xs0 - xd0)
                c1 = g * (xs1 - xd1)
                b0 = plsc.bitcast(c0, jnp.int32)
                b1 = plsc.bitcast(c1, jnp.int32)
                lo = lax.shift_right_logical(b0 + _RND, 16)
                hi = (b1 + _RND) & _MASKHI
                cbuf[pl.ds(o, _L)] = plsc.bitcast(lo | hi, jnp.float32)

        def chunk1(ci2, _):
            ci = ci2 * 2
            issue_in1(ci + 1, 1)
            wait_in1(ci, 0)

            @pl.when(ci2 >= 1)
            def _():
                pltpu.make_async_copy(cb0, cur_hbm.at[pl.ds(coff1(ci), C)],
                                      outsem.at[0]).wait()

            compute1(0)
            pltpu.async_copy(cb0, cur_hbm.at[pl.ds(coff1(ci), C)], outsem.at[0])

            @pl.when(ci + 2 < NCHUNK1)
            def _():
                issue_in1(ci + 2, 0)

            wait_in1(ci + 1, 1)

            @pl.when(ci2 >= 1)
            def _():
                pltpu.make_async_copy(cb1, cur_hbm.at[pl.ds(coff1(ci + 1), C)],
                                      outsem.at[1]).wait()

            compute1(1)
            pltpu.async_copy(cb1, cur_hbm.at[pl.ds(coff1(ci + 1), C)],
                             outsem.at[1])
            return 0

        lax.fori_loop(0, NCHUNK1 // 2, chunk1, 0)
        pltpu.make_async_copy(cb0, cur_hbm.at[pl.ds(coff1(NCHUNK1 - 2), C)],
                              outsem.at[0]).wait()
        pltpu.make_async_copy(cb1, cur_hbm.at[pl.ds(coff1(NCHUNK1 - 1), C)],
                              outsem.at[1]).wait()
        plsc.subcore_barrier()
        scope1.__exit__(None, None, None)
        scope2 = jax.named_scope("ph2_scatter")
        scope2.__enter__()

        issue_in2(0, 0)
        zeros = jnp.zeros((_L,), jnp.float32)

        @plsc.parallel_loop(0, N // _L, unroll=8)
        def _(i):
            xacc[pl.ds(i * _L, _L)] = zeros

        shv = jnp.zeros((_L,), jnp.int32) + half * 16

        def compute2(slot):
            sbuf, dbuf, cbuf = sb[slot], db[slot], tb[slot]

            @plsc.parallel_loop(0, VPC, unroll=4)
            def _(i):
                o = i * _L
                sv = sbuf[pl.ds(o, _L)]
                dv = dbuf[pl.ds(o, _L)]
                cp = plsc.bitcast(cbuf[pl.ds(o, _L)], jnp.int32)
                cv = plsc.bitcast(lax.shift_right_arithmetic(cp, shv) << 16,
                                  jnp.float32)
                plsc.addupdate_scatter(xacc, [sv], cv)
                plsc.addupdate_scatter(xacc, [dv], -cv)

        def chunk2(ci2, _):
            ci = ci2 * 2
            issue_in2(ci + 1, 1)
            wait_in2(ci, 0)
            compute2(0)

            @pl.when(ci + 2 < NCHUNK2)
            def _():
                issue_in2(ci + 2, 0)

            wait_in2(ci + 1, 1)
            compute2(1)
            return 0

        lax.fori_loop(0, NCHUNK2 // 2, chunk2, 0)

        scope2.__exit__(None, None, None)
        scope3 = jax.named_scope("ph3_combine")
        scope3.__enter__()
        pltpu.sync_copy(xacc, part_hbm.at[pl.ds(pl.multiple_of(wrow * N, 8), N)])
        plsc.subcore_barrier()

        nstart = c8 * NP8 - (c8 * NP8) % 16
        nstart = pl.multiple_of(lax.min(nstart, N - SIZE), 16)
        xo = 0
        po = (SIZE, 2 * SIZE)

        def ppart(j):
            return part_hbm.at[pl.ds(
                pl.multiple_of((bg * 8 + j) * N + nstart, 16), SIZE)]

        xslice = x_hbm.at[pl.ds(pl.multiple_of(bg * N + nstart, 16), SIZE)]
        pltpu.async_copy(xslice, xacc.at[pl.ds(xo, SIZE)], insem.at[0])
        pltpu.async_copy(ppart(0), xacc.at[pl.ds(po[0], SIZE)], outsem.at[0])
        pltpu.make_async_copy(xslice, xacc.at[pl.ds(xo, SIZE)],
                              insem.at[0]).wait()
        for j in range(8):
            slot = j % 2
            if j + 1 < 8:
                pltpu.async_copy(ppart(j + 1), xacc.at[pl.ds(po[1 - slot], SIZE)],
                                 outsem.at[1 - slot])
            pltpu.make_async_copy(ppart(j), xacc.at[pl.ds(po[slot], SIZE)],
                                  outsem.at[slot]).wait()
            pj = po[slot]

            @plsc.parallel_loop(0, SIZE // _L, unroll=4)
            def _(i, _pj=pj):
                o = i * _L
                xacc[pl.ds(xo + o, _L)] = (
                    xacc[pl.ds(xo + o, _L)] - _LR * xacc[pl.ds(_pj + o, _L)])

        pltpu.sync_copy(xacc.at[pl.ds(xo, SIZE)],
                        out_hbm.at[pl.ds(pl.multiple_of(bg * N + nstart, 16), SIZE)])
        scope3.__exit__(None, None, None)

    return run


def kernel(x, theta, edge_index):
    B, N = x.shape
    E = edge_index.shape[1]
    src = edge_index[0].astype(jnp.int32)
    dst = edge_index[1].astype(jnp.int32)
    th = theta.reshape(E)
    xf = x.reshape(B * N)
    xb = lax.bitcast_convert_type(x.astype(jnp.bfloat16), jnp.uint16)
    xb = xb.astype(jnp.uint32)
    xp = jnp.concatenate([xb[0] | (xb[1] << 16), xb[2] | (xb[3] << 16)])
    xp = lax.bitcast_convert_type(xp, jnp.float32)
    out_flat, _, _ = _build(B, N, E)(xf, xp, src, dst, th)
    return out_flat.reshape(B, N)
